# Initial kernel scaffold; baseline (speedup 1.0000x reference)
import jax
import jax.numpy as jnp
from jax import lax
from jax.experimental import pallas as pl
from jax.experimental.pallas import tpu as pltpu
from jax.experimental.pallas import tpu_sc as plsc

_NC = 2
_NS = 16
_NW = _NC * _NS
_LANES = 16

_F = 26
_V = 100000
_D = 32
_SUB = 1024


def _body(x_hbm, tab_hbm, out_hbm, idx_v, rows_v, sem):
    wid = lax.axis_index("s") * _NC + lax.axis_index("c")
    rows_total = 16384 * _F
    rows_per_w = rows_total // _NW
    n_sub = rows_per_w // _SUB

    def step(t, carry):
        base = wid * rows_per_w + t * _SUB
        pltpu.sync_copy(x_hbm.at[pl.ds(base, _SUB)], idx_v)

        def fix(i, c):
            pos = base + i * _LANES + lax.iota(jnp.int32, _LANES)
            j = lax.rem(pos, _F)
            sl = pl.ds(i * _LANES, _LANES)
            idx_v[sl] = idx_v[sl] + j * _V
            return c

        lax.fori_loop(0, _SUB // _LANES, fix, 0)

        pltpu.async_copy(tab_hbm.at[idx_v], rows_v, sem).wait()
        pltpu.sync_copy(rows_v, out_hbm.at[pl.ds(base, _SUB)])
        return carry

    lax.fori_loop(0, n_sub, step, 0)


def kernel(x_cat, tables):
    B, F = x_cat.shape
    _, V, D = tables.shape
    x_flat = x_cat.reshape(-1)
    tab_flat = tables.reshape(F * V, D)
    rows_total = B * F

    k = pl.kernel(
        _body,
        out_type=jax.ShapeDtypeStruct((rows_total, D), jnp.float32),
        mesh=plsc.VectorSubcoreMesh(core_axis_name="c", subcore_axis_name="s"),
        scratch_types=[
            pltpu.VMEM((_SUB,), jnp.int32),
            pltpu.VMEM((_SUB, _D), jnp.float32),
            pltpu.SemaphoreType.DMA,
        ],
    )
    out = k(x_flat, tab_flat)
    return out.reshape(B, F * D)


# speedup vs baseline: 1.2014x; 1.2014x over previous
import jax
import jax.numpy as jnp
from jax import lax
from jax.experimental import pallas as pl
from jax.experimental.pallas import tpu as pltpu
from jax.experimental.pallas import tpu_sc as plsc

_NC = 2
_NS = 16
_NW = _NC * _NS
_LANES = 16

_F = 26
_V = 100000
_D = 32
_SUB = 1024


def _body(x_hbm, tab_hbm, out_hbm, idx_v, rows_v, sem):
    wid = lax.axis_index("s") * _NC + lax.axis_index("c")
    rows_total = 16384 * _F
    rows_per_w = rows_total // _NW
    n_sub = rows_per_w // _SUB

    def step(t, carry):
        base = wid * rows_per_w + t * _SUB
        pltpu.sync_copy(x_hbm.at[pl.ds(base, _SUB)], idx_v)

        def fix(i, c):
            pos = base + i * _LANES + lax.iota(jnp.int32, _LANES)
            j = lax.rem(pos, _F)
            sl = pl.ds(i * _LANES, _LANES)
            idx_v[sl] = idx_v[sl] + j * _V
            return c

        lax.fori_loop(0, _SUB // _LANES, fix, 0)

        pltpu.async_copy(tab_hbm.at[idx_v], rows_v, sem).wait()
        pltpu.sync_copy(rows_v, out_hbm.at[pl.ds(base, _SUB)])
        return carry

    lax.fori_loop(0, n_sub, step, 0)


def kernel(x_cat, tables):
    B, F = x_cat.shape
    _, V, D = tables.shape
    x_flat = x_cat.reshape(-1)
    tab_flat = tables.reshape(F * V, D)
    rows_total = B * F

    k = pl.kernel(
        _body,
        out_type=jax.ShapeDtypeStruct((rows_total, D), jnp.float32),
        mesh=plsc.VectorSubcoreMesh(core_axis_name="c", subcore_axis_name="s"),
        scratch_types=[
            pltpu.VMEM((_SUB,), jnp.int32),
            pltpu.VMEM((_SUB, _D), jnp.float32),
            pltpu.SemaphoreType.DMA,
        ],
        compiler_params=pltpu.CompilerParams(use_tc_tiling_on_sc=False),
    )
    out = k(x_flat, tab_flat)
    return out.reshape(B, F * D)


# speedup vs baseline: 1.2162x; 1.0123x over previous
import jax
import jax.numpy as jnp
from jax import lax
from jax.experimental import pallas as pl
from jax.experimental.pallas import tpu as pltpu
from jax.experimental.pallas import tpu_sc as plsc

_NC = 2
_NS = 16
_NW = _NC * _NS
_LANES = 16

_F = 26
_V = 100000
_D = 32
_B = 16384

_RPW = _B * _F // _NW
_SUB = 1024
_NSTEP = _RPW // _SUB
_NBUF = 3
_VPC = _SUB // _LANES


def _body(x_hbm, tab_hbm, out_hbm, idx_v, rows_v, *sems):
    sg = sems[:_NBUF]
    ss = sems[_NBUF:]
    wid = lax.axis_index("s") * _NC + lax.axis_index("c")
    base_w = wid * _RPW

    pltpu.sync_copy(x_hbm.at[pl.ds(base_w, _RPW)], idx_v)

    def fix_chunk(c):
        init = (c * _SUB) % _F

        def fxb(i, jv):
            sl = pl.ds((c * _VPC + i) * _LANES, _LANES)
            idx_v[sl] = idx_v[sl] + jv * _V
            jv = jv + _LANES
            return jnp.where(jv >= _F, jv - _F, jv)

        jv0 = lax.rem(lax.iota(jnp.int32, _LANES) + init, _F)
        lax.fori_loop(0, _VPC, fxb, jv0)

    def fire_gather(t):
        return pltpu.async_copy(
            tab_hbm.at[idx_v.at[pl.ds(t * _SUB, _SUB)]],
            rows_v.at[t % _NBUF],
            sg[t % _NBUF],
        )

    def fire_store(t):
        return pltpu.async_copy(
            rows_v.at[t % _NBUF],
            out_hbm.at[pl.ds(base_w + t * _SUB, _SUB)],
            ss[t % _NBUF],
        )

    hg = {}
    hs = {}
    for t in range(_NBUF):
        fix_chunk(t)
        hg[t] = fire_gather(t)
    for t in range(_NSTEP):
        if t + _NBUF < _NSTEP:
            fix_chunk(t + _NBUF)
        hg[t].wait()
        hs[t] = fire_store(t)
        if t + _NBUF < _NSTEP:
            hs[t].wait()
            hg[t + _NBUF] = fire_gather(t + _NBUF)
    for t in range(max(0, _NSTEP - _NBUF), _NSTEP):
        hs[t].wait()


def kernel(x_cat, tables):
    B, F = x_cat.shape
    _, V, D = tables.shape
    x_flat = x_cat.reshape(-1)
    tab_flat = tables.reshape(F * V, D)
    rows_total = B * F

    k = pl.kernel(
        _body,
        out_type=jax.ShapeDtypeStruct((rows_total, D), jnp.float32),
        mesh=plsc.VectorSubcoreMesh(core_axis_name="c", subcore_axis_name="s"),
        scratch_types=[
            pltpu.VMEM((_RPW,), jnp.int32),
            pltpu.VMEM((_NBUF, _SUB, _D), jnp.float32),
        ] + [pltpu.SemaphoreType.DMA] * (2 * _NBUF),
        compiler_params=pltpu.CompilerParams(use_tc_tiling_on_sc=False),
    )
    out = k(x_flat, tab_flat)
    return out.reshape(B, F * D)


# speedup vs baseline: 3.6495x; 3.0008x over previous
import jax
import jax.numpy as jnp
from jax import lax
from jax.experimental import pallas as pl
from jax.experimental.pallas import tpu as pltpu
from jax.experimental.pallas import tpu_sc as plsc

_NC = 2
_NS = 16
_NW = _NC * _NS
_LANES = 16

_F = 26
_V = 100000
_D = 32
_B = 16384
_CH = 8192


def _body(xT_hbm, t3_hbm, outT_hbm, row_v, idx_v, out_v, sem_row, sem_idx):
    w = lax.axis_index("s") * _NC + lax.axis_index("c")

    def pair(k, carry):
        jd = w + _NW * k
        h_row = pltpu.async_copy(t3_hbm.at[jd], row_v, sem_row)
        h_idx = pltpu.async_copy(xT_hbm.at[k], idx_v, sem_idx)
        h_row.wait()
        h_idx.wait()
        for c in range(_B // _CH):
            def gath(i, _):
                iv = idx_v[pl.ds(c * _CH + i * _LANES, _LANES)]
                out_v[pl.ds(i * _LANES, _LANES)] = plsc.load_gather(row_v, [iv])
                return 0

            lax.fori_loop(0, _CH // _LANES, gath, 0, unroll=8)
            pltpu.sync_copy(out_v, outT_hbm.at[jd, pl.ds(c * _CH, _CH)])
        return carry

    lax.fori_loop(0, _F, pair, 0)


def kernel(x_cat, tables):
    B, F = x_cat.shape
    _, V, D = tables.shape
    xT = x_cat.T
    t3 = tables.transpose(0, 2, 1).reshape(F * D, V)

    k = pl.kernel(
        _body,
        out_type=jax.ShapeDtypeStruct((F * D, B), jnp.float32),
        mesh=plsc.VectorSubcoreMesh(core_axis_name="c", subcore_axis_name="s"),
        scratch_types=[
            pltpu.VMEM((V,), jnp.float32),
            pltpu.VMEM((B,), jnp.int32),
            pltpu.VMEM((_CH,), jnp.float32),
            pltpu.SemaphoreType.DMA,
            pltpu.SemaphoreType.DMA,
        ],
        compiler_params=pltpu.CompilerParams(
            use_tc_tiling_on_sc=True, needs_layout_passes=False
        ),
    )
    outT = k(xT, t3)
    return outT.T


# speedup vs baseline: 6.6927x; 1.8339x over previous
import jax
import jax.numpy as jnp
from jax import lax
from jax.experimental import pallas as pl
from jax.experimental.pallas import tpu as pltpu
from jax.experimental.pallas import tpu_sc as plsc

_NC = 2
_NS = 16
_NW = _NC * _NS
_LANES = 16

_F = 26
_V = 100000
_D = 32
_B = 16384
_CH = 4096


def _body(xT_hbm, t3_hbm, outT_hbm, row_v, idx_v, out_v, sem_row, sem_idx,
          sem_o0, sem_o1):
    w = lax.axis_index("s") * _NC + lax.axis_index("c")
    sem_o = [sem_o0, sem_o1]
    n_ch = _B // _CH

    def pair(k, carry):
        jd = w + _NW * k
        h_row = pltpu.async_copy(t3_hbm.at[jd], row_v, sem_row)
        h_idx = pltpu.async_copy(xT_hbm.at[k], idx_v, sem_idx)
        h_row.wait()
        h_idx.wait()
        stores = [None, None]
        for c in range(n_ch):
            @plsc.parallel_loop(0, _CH // _LANES, unroll=8)
            def gath(i):
                iv = idx_v[pl.ds(c * _CH + i * _LANES, _LANES)]
                out_v[c % 2, pl.ds(i * _LANES, _LANES)] = (
                    plsc.load_gather(row_v, [iv]))

            if stores[c % 2] is not None:
                stores[c % 2].wait()
            stores[c % 2] = pltpu.async_copy(
                out_v.at[c % 2],
                outT_hbm.at[jd, pl.ds(c * _CH, _CH)],
                sem_o[c % 2],
            )
        stores[(n_ch - 2) % 2].wait()
        stores[(n_ch - 1) % 2].wait()
        return carry

    lax.fori_loop(0, _F, pair, 0)


def kernel(x_cat, tables):
    B, F = x_cat.shape
    _, V, D = tables.shape
    xT = x_cat.T
    t3 = tables.transpose(0, 2, 1).reshape(F * D, V)

    k = pl.kernel(
        _body,
        out_type=jax.ShapeDtypeStruct((F * D, B), jnp.float32),
        mesh=plsc.VectorSubcoreMesh(core_axis_name="c", subcore_axis_name="s"),
        scratch_types=[
            pltpu.VMEM((V,), jnp.float32),
            pltpu.VMEM((B,), jnp.int32),
            pltpu.VMEM((2, _CH), jnp.float32),
            pltpu.SemaphoreType.DMA,
            pltpu.SemaphoreType.DMA,
            pltpu.SemaphoreType.DMA,
            pltpu.SemaphoreType.DMA,
        ],
        compiler_params=pltpu.CompilerParams(
            use_tc_tiling_on_sc=True, needs_layout_passes=False
        ),
    )
    outT = k(xT, t3)
    return outT.T


# speedup vs baseline: 6.6954x; 1.0004x over previous
import jax
import jax.numpy as jnp
from jax import lax
from jax.experimental import pallas as pl
from jax.experimental.pallas import tpu as pltpu
from jax.experimental.pallas import tpu_sc as plsc

_NC = 2
_NS = 16
_NW = _NC * _NS
_LANES = 16

_F = 26
_V = 100000
_D = 32
_B = 16384
_CH = 4096


def _body(xT_hbm, t3_hbm, outT_hbm, row_v, idx_v, out_v, sem_row, sem_idx,
          sem_o0, sem_o1):
    w = lax.axis_index("s") * _NC + lax.axis_index("c")
    sem_o = [sem_o0, sem_o1]
    n_ch = _B // _CH

    def pair(k, carry):
        jd = w + _NW * k
        h_row = pltpu.async_copy(t3_hbm.at[jd], row_v, sem_row)
        h_idx = pltpu.async_copy(xT_hbm.at[k], idx_v, sem_idx)
        h_row.wait()
        h_idx.wait()
        stores = [None, None]
        for c in range(n_ch):
            @plsc.parallel_loop(0, _CH // _LANES, unroll=16)
            def gath(i):
                iv = idx_v[pl.ds(c * _CH + i * _LANES, _LANES)]
                out_v[c % 2, pl.ds(i * _LANES, _LANES)] = (
                    plsc.load_gather(row_v, [iv]))

            if stores[c % 2] is not None:
                stores[c % 2].wait()
            stores[c % 2] = pltpu.async_copy(
                out_v.at[c % 2],
                outT_hbm.at[jd, pl.ds(c * _CH, _CH)],
                sem_o[c % 2],
            )
        stores[(n_ch - 2) % 2].wait()
        stores[(n_ch - 1) % 2].wait()
        return carry

    lax.fori_loop(0, _F, pair, 0)


def kernel(x_cat, tables):
    B, F = x_cat.shape
    _, V, D = tables.shape
    xT = x_cat.T
    t3 = tables.transpose(0, 2, 1).reshape(F * D, V)

    k = pl.kernel(
        _body,
        out_type=jax.ShapeDtypeStruct((F * D, B), jnp.float32),
        mesh=plsc.VectorSubcoreMesh(core_axis_name="c", subcore_axis_name="s"),
        scratch_types=[
            pltpu.VMEM((V,), jnp.float32),
            pltpu.VMEM((B,), jnp.int32),
            pltpu.VMEM((2, _CH), jnp.float32),
            pltpu.SemaphoreType.DMA,
            pltpu.SemaphoreType.DMA,
            pltpu.SemaphoreType.DMA,
            pltpu.SemaphoreType.DMA,
        ],
        compiler_params=pltpu.CompilerParams(
            use_tc_tiling_on_sc=True, needs_layout_passes=False
        ),
    )
    outT = k(xT, t3)
    return outT.T


# speedup vs baseline: 6.9538x; 1.0386x over previous
import jax
import jax.numpy as jnp
from jax import lax
from jax.experimental import pallas as pl
from jax.experimental.pallas import tpu as pltpu
from jax.experimental.pallas import tpu_sc as plsc

_NC = 2
_NS = 16
_NW = _NC * _NS
_LANES = 16

_F = 26
_V = 100000
_D = 32
_B = 16384
_CH = 4096
_VA = (_V // 128) * 128


def _body(xT_hbm, t3_hbm, outT_hbm, row_v, idx_v, out_v, sem_q0, sem_q1,
          sem_q2, sem_q3, sem_idx, sem_o0, sem_o1):
    sem_q = [sem_q0, sem_q1, sem_q2, sem_q3]
    w = lax.axis_index("s") * _NC + lax.axis_index("c")
    sem_o = [sem_o0, sem_o1]
    n_ch = _B // _CH

    zero16 = jnp.zeros((_LANES,), jnp.int32)

    def pair(k, carry):
        jd = w + _NW * k
        qb = [0, 25088, 50176, 75264, _V]
        hq = [
            pltpu.async_copy(
                t3_hbm.at[pl.ds(jd, 1), pl.ds(qb[q], qb[q + 1] - qb[q])],
                row_v.at[:, pl.ds(qb[q], qb[q + 1] - qb[q])],
                sem_q[q],
            )
            for q in range(4)
        ]
        h_idx = pltpu.async_copy(xT_hbm.at[k], idx_v, sem_idx)
        for h in hq:
            h.wait()
        h_idx.wait()
        stores = [None, None]
        for c in range(n_ch):
            @plsc.parallel_loop(0, _CH // _LANES, unroll=16)
            def gath(i):
                iv = idx_v[pl.ds(c * _CH + i * _LANES, _LANES)]
                out_v[c % 2, pl.ds(i * _LANES, _LANES)] = (
                    plsc.load_gather(row_v, [zero16, iv]))

            if stores[c % 2] is not None:
                stores[c % 2].wait()
            stores[c % 2] = pltpu.async_copy(
                out_v.at[c % 2],
                outT_hbm.at[jd, pl.ds(c * _CH, _CH)],
                sem_o[c % 2],
            )
        stores[(n_ch - 2) % 2].wait()
        stores[(n_ch - 1) % 2].wait()
        return carry

    lax.fori_loop(0, _F, pair, 0)


def kernel(x_cat, tables):
    B, F = x_cat.shape
    _, V, D = tables.shape
    xT = x_cat.T
    t3 = tables.transpose(0, 2, 1).reshape(F * D, V)

    k = pl.kernel(
        _body,
        out_type=jax.ShapeDtypeStruct((F * D, B), jnp.float32),
        mesh=plsc.VectorSubcoreMesh(core_axis_name="c", subcore_axis_name="s"),
        scratch_types=[
            pltpu.VMEM((1, V), jnp.float32),
            pltpu.VMEM((B,), jnp.int32),
            pltpu.VMEM((2, _CH), jnp.float32),
        ] + [pltpu.SemaphoreType.DMA] * 7,
        compiler_params=pltpu.CompilerParams(
            use_tc_tiling_on_sc=True, needs_layout_passes=False
        ),
    )
    outT = k(xT, t3)
    return outT.T


# speedup vs baseline: 7.7132x; 1.1092x over previous
import jax
import jax.numpy as jnp
from jax import lax
from jax.experimental import pallas as pl
from jax.experimental.pallas import tpu as pltpu
from jax.experimental.pallas import tpu_sc as plsc

_NC = 2
_NS = 16
_NW = _NC * _NS
_LANES = 16

_F = 26
_V = 100000
_D = 32
_B = 16384
_CH = 4096
_VA = (_V // 128) * 128


def _body(xT_hbm, t3_hbm, outT_hbm, row_v, idx_v, out_v, sem_q0,
          sem_q1, sem_q2, sem_q3, sem_o0, sem_o1):
    sem_q = [sem_q0, sem_q1, sem_q2, sem_q3]
    w = lax.axis_index("s") * _NC + lax.axis_index("c")
    sem_o = [sem_o0, sem_o1]
    n_ch = _B // _CH

    zero16 = jnp.zeros((_LANES,), jnp.int32)

    def pair(i, prev_j):
        jd = _F * w + i
        j = jd // _D

        @pl.when(j != prev_j)
        def _():
            pltpu.sync_copy(xT_hbm.at[j], idx_v)
        qb = [0, 25088, 50176, 75264, _V]
        hq = [
            pltpu.async_copy(
                t3_hbm.at[pl.ds(jd, 1), pl.ds(qb[q], qb[q + 1] - qb[q])],
                row_v.at[:, pl.ds(qb[q], qb[q + 1] - qb[q])],
                sem_q[q],
            )
            for q in range(4)
        ]
        for h in hq:
            h.wait()
        stores = [None, None]
        for c in range(n_ch):
            @plsc.parallel_loop(0, _CH // _LANES, unroll=16)
            def gath(g):
                iv = idx_v[pl.ds(c * _CH + g * _LANES, _LANES)]
                out_v[c % 2, pl.ds(g * _LANES, _LANES)] = (
                    plsc.load_gather(row_v, [zero16, iv]))

            if stores[c % 2] is not None:
                stores[c % 2].wait()
            stores[c % 2] = pltpu.async_copy(
                out_v.at[c % 2],
                outT_hbm.at[jd, pl.ds(c * _CH, _CH)],
                sem_o[c % 2],
            )
        stores[(n_ch - 2) % 2].wait()
        stores[(n_ch - 1) % 2].wait()
        return j

    lax.fori_loop(0, _F, pair, jnp.int32(-1))


def kernel(x_cat, tables):
    B, F = x_cat.shape
    _, V, D = tables.shape
    xT = x_cat.T
    t3 = tables.transpose(0, 2, 1).reshape(F * D, V)

    k = pl.kernel(
        _body,
        out_type=jax.ShapeDtypeStruct((F * D, B), jnp.float32),
        mesh=plsc.VectorSubcoreMesh(core_axis_name="c", subcore_axis_name="s"),
        scratch_types=[
            pltpu.VMEM((1, V), jnp.float32),
            pltpu.VMEM((B,), jnp.int32),
            pltpu.VMEM((2, _CH), jnp.float32),
        ] + [pltpu.SemaphoreType.DMA] * 6,
        compiler_params=pltpu.CompilerParams(
            use_tc_tiling_on_sc=True, needs_layout_passes=False
        ),
    )
    outT = k(xT, t3)
    return outT.T
